# baseline (speedup 1.0000x reference)
import functools
import math

import jax
import jax.numpy as jnp
from jax import lax
from jax.experimental import pallas as pl
from jax.experimental.pallas import tpu as pltpu
from jax.experimental.pallas import tpu_sc as plsc

A = 26
D = 16
C = 100000
B = 16384
TD = A * D

NC = 2
NS = 16
NW = NC * NS
SPW = B // NW
CH = 64
NCHUNK = SPW // CH

_LOG2PI = math.log(2.0 * math.pi)


def _sc_body(z_hbm, sldj_hbm, means_hbm, labels_hbm, out_hbm,
             z_v, lab_v, mu_v, idx_v, sldj_v, out_v, sem):
    wid = lax.axis_index("s") * NC + lax.axis_index("c")

    iota16 = lax.iota(jnp.int32, 16)

    def chunk_body(g, carry):
        accq, accs = carry
        base = wid * SPW + g * CH

        pltpu.sync_copy(z_hbm.at[pl.ds(base, CH)], z_v)
        pltpu.sync_copy(labels_hbm.at[pl.ds(base, CH)], lab_v)
        pltpu.sync_copy(sldj_hbm.at[pl.ds(base, CH)], sldj_v)

        for i in range(A):
            for t in range(CH // 16):
                rows = iota16 + t * 16
                cols = jnp.full((16,), i, jnp.int32)
                labs = plsc.load_gather(lab_v, [rows, cols])
                idx_v[i, pl.ds(t * 16, 16)] = labs

        copies = [
            pltpu.async_copy(means_hbm.at[i].at[idx_v.at[i]], mu_v.at[i], sem)
            for i in range(A)
        ]
        for c in copies:
            c.wait()

        def sample_body(s, acc):
            zrows = jnp.full((16,), s, jnp.int32)
            zcols = iota16 * A
            for i in range(A):
                vz = plsc.load_gather(z_v, [zrows, zcols + i])
                vmu = mu_v[i, s, :]
                dz = vz - vmu
                acc = acc + dz * dz
            return acc

        accq = lax.fori_loop(0, CH, sample_body, accq)

        for t in range(CH // 16):
            accs = accs + sldj_v[pl.ds(t * 16, 16)]
        return accq, accs

    zero = jnp.zeros((16,), jnp.float32)
    accq, accs = lax.fori_loop(0, NCHUNK, chunk_body, (zero, zero))

    out_v[...] = accq * 0.5 - accs
    pltpu.sync_copy(out_v, out_hbm.at[wid])


@jax.jit
def _sc_partials(z, sldj, means, labels):
    mesh = plsc.VectorSubcoreMesh(core_axis_name="c", subcore_axis_name="s")
    run = functools.partial(
        pl.kernel,
        mesh=mesh,
        out_type=jax.ShapeDtypeStruct((NW, 16), jnp.float32),
        compiler_params=pltpu.CompilerParams(
            needs_layout_passes=False, use_tc_tiling_on_sc=False),
        scratch_types=[
            pltpu.VMEM((CH, TD), jnp.float32),
            pltpu.VMEM((CH, A), jnp.int32),
            pltpu.VMEM((A, CH, D), jnp.float32),
            pltpu.VMEM((A, CH), jnp.int32),
            pltpu.VMEM((CH,), jnp.float32),
            pltpu.VMEM((16,), jnp.float32),
            pltpu.SemaphoreType.DMA,
        ],
    )(_sc_body)
    return run(z, sldj, means, labels)


def kernel(z, sldj, means, labels):
    partials = _sc_partials(z, sldj, means, labels)
    return jnp.sum(partials) / B + 0.5 * TD * _LOG2PI


# speedup vs baseline: 8.2420x; 8.2420x over previous
import functools
import math

import jax
import jax.numpy as jnp
from jax import lax
from jax.experimental import pallas as pl
from jax.experimental.pallas import tpu as pltpu
from jax.experimental.pallas import tpu_sc as plsc

A = 26
D = 16
C = 100000
B = 16384
TD = A * D

NC = 2
NS = 16
NW = NC * NS
UNITS = TD
UPW = UNITS // NW
SB = 8192

_LOG2PI = math.log(2.0 * math.pi)


def _sc_body(zt_hbm, sldj_hbm, meanst_hbm, labelst_hbm, out_hbm,
             row_v, z_v, lab_v, sldj_v, out_v, semz, semr):
    wid = lax.axis_index("s") * NC + lax.axis_index("c")
    sid = lax.axis_index("s")

    def unit_body(j, carry):
        acc, prev_i = carry
        jr = lax.rem(j + sid, UPW)
        u = wid * UPW + jr
        i = u // D
        d = u % D
        cz = pltpu.async_copy(zt_hbm.at[d * A + i, :], z_v, semz)
        cr = pltpu.async_copy(meanst_hbm.at[i, d, :], row_v, semr)

        def process_half(h, a):
            def group_body(g, aa):
                for k in range(8):
                    off = g * 128 + k * 16
                    lab = lab_v[pl.ds(off, 16)]
                    mu = plsc.load_gather(row_v, [lab])
                    vz = z_v[pl.ds(h * SB + off, 16)]
                    t = vz - mu
                    aa = aa + t * t
                return aa

            return lax.fori_loop(0, SB // 128, group_body, a)

        par = j % 2
        h_first = par
        h_second = 1 - par

        @pl.when(jnp.logical_or(j == 0, i != prev_i))
        def _():
            pltpu.sync_copy(labelst_hbm.at[i, pl.ds(h_first * SB, SB)], lab_v)

        cr.wait()
        cz.wait()
        acc = process_half(h_first, acc)
        pltpu.sync_copy(labelst_hbm.at[i, pl.ds(h_second * SB, SB)], lab_v)
        acc = process_half(h_second, acc)
        return acc, i

    zero = jnp.zeros((16,), jnp.float32)
    accq, _ = lax.fori_loop(0, UPW, unit_body, (zero, jnp.int32(-1)))

    pltpu.sync_copy(sldj_hbm.at[pl.ds(wid * (B // NW), B // NW)], sldj_v)

    def sldj_body(t, a):
        return a + sldj_v[pl.ds(t * 16, 16)]

    accs = lax.fori_loop(0, B // (NW * 16), sldj_body, zero)

    out_v[...] = accq * 0.5 - accs
    pltpu.sync_copy(out_v, out_hbm.at[wid])


@jax.jit
def _sc_partials(zt, sldj, meanst, labelst):
    mesh = plsc.VectorSubcoreMesh(core_axis_name="c", subcore_axis_name="s")
    run = functools.partial(
        pl.kernel,
        mesh=mesh,
        out_type=jax.ShapeDtypeStruct((NW, 16), jnp.float32),
        compiler_params=pltpu.CompilerParams(needs_layout_passes=False),
        scratch_types=[
            pltpu.VMEM((C,), jnp.float32),
            pltpu.VMEM((B,), jnp.float32),
            pltpu.VMEM((SB,), jnp.int32),
            pltpu.VMEM((B // NW,), jnp.float32),
            pltpu.VMEM((16,), jnp.float32),
            pltpu.SemaphoreType.DMA,
            pltpu.SemaphoreType.DMA,
        ],
    )(_sc_body)
    return run(zt, sldj, meanst, labelst)


def kernel(z, sldj, means, labels):
    zt = z.T
    meanst = jnp.transpose(means, (0, 2, 1))
    labelst = labels.T
    partials = _sc_partials(zt, sldj, meanst, labelst)
    return jnp.sum(partials) / B + 0.5 * TD * _LOG2PI
